# baseline (speedup 1.0000x reference)
import jax
import jax.numpy as jnp
from jax import lax
from jax.experimental import pallas as pl
from jax.experimental.pallas import tpu as pltpu
from jax.experimental.pallas import tpu_sc as plsc

_N = 1000000
_H = 16
_C = 6
_B = 16384

_NC = 2
_NS = 16
_NW = _NC * _NS
_BPW = _B // _NW
_NBLK = _BPW // 16


def _body(r_hbm, w_hbm, tab, out_hbm, r_v, rows, w_v, out_t, sem):
  wid = lax.axis_index("s") * _NC + lax.axis_index("c")
  base = wid * _BPW
  pltpu.sync_copy(r_hbm.at[wid], r_v)
  pltpu.sync_copy(w_hbm, w_v)

  iota = lax.broadcasted_iota(jnp.int32, (16,), 0)

  def issue_block(blk):
    j0 = blk * 16
    r_vec = r_v[pl.ds(j0, 16)]
    for k in range(16):
      r_k = jnp.sum(jnp.where(iota == k, r_vec, 0))
      pltpu.async_copy(tab.at[r_k], rows.at[j0 + k], sem)

  def drain_block():
    for _ in range(16):
      pltpu.make_async_copy(tab.at[0], rows.at[0], sem).wait()

  def compute_block(blk):
    j0 = blk * 16
    rowv = j0 + iota
    acc = [w_v[_H * _C + c, :] for c in range(_C)]
    for h in range(_H):
      hv = jnp.full((16,), h, dtype=jnp.int32)
      c16 = plsc.load_gather(rows, [rowv, hv])
      for c in range(_C):
        acc[c] = acc[c] + c16 * w_v[h * _C + c, :]
    for c in range(_C):
      out_t[c, pl.ds(j0, 16)] = acc[c]

  issue_block(0)

  def body(blk, _):
    issue_block(blk + 1)
    drain_block()
    compute_block(blk)
    return 0

  lax.fori_loop(0, _NBLK - 1, body, 0)
  drain_block()
  compute_block(_NBLK - 1)

  pltpu.sync_copy(out_t, out_hbm.at[:, pl.ds(base, _BPW)])


_sc_partial = pl.kernel(
    _body,
    out_type=jax.ShapeDtypeStruct((_C, _B), jnp.float32),
    mesh=plsc.VectorSubcoreMesh(core_axis_name="c", subcore_axis_name="s"),
    compiler_params=pltpu.CompilerParams(needs_layout_passes=False),
    scratch_types=[
        pltpu.VMEM((_BPW,), jnp.int32),
        pltpu.VMEM((_BPW, _H), jnp.float32),
        pltpu.VMEM((_H * _C + _C, 16), jnp.float32),
        pltpu.VMEM((_C, _BPW), jnp.float32),
        pltpu.SemaphoreType.DMA,
    ],
)


@jax.jit
def kernel(X_batch, U, V, W, b):
  x0 = X_batch[:, 0].astype(jnp.int32)
  x1 = X_batch[:, 1].astype(jnp.int32)
  ur = x0.reshape(_NW, _BPW)
  vr = x1.reshape(_NW, _BPW)
  wu = jnp.broadcast_to(
      jnp.concatenate([W[:_H].reshape(-1), jnp.zeros((_C,), jnp.float32)]
                      )[:, None], (_H * _C + _C, 16))
  wv = jnp.broadcast_to(
      jnp.concatenate([W[_H:].reshape(-1), b])[:, None], (_H * _C + _C, 16))
  part_u = _sc_partial(ur, wu, U)
  part_v = _sc_partial(vr, wv, V)
  return (part_u + part_v).T
